# baseline (speedup 1.0000x reference)
import jax
import jax.numpy as jnp
from jax import lax
from jax.experimental import pallas as pl
from jax.experimental.pallas import tpu as pltpu

TEMP = 0.07
HARD_NEG_RATIO = 0.2
NUM_CLASSES = 10
CB = 128
G = 256
CHUNK_L = 1024
BISECT_ITERS = 32
UNROLL = 8


def _loss_kernel(hf_ref, hmT_ref, hm_ref, oh_ref, ohT_ref, lab_ref, out_ref,
                 keysT_ref, rowlse_ref, colmax_ref, colsum_ref, lo_ref):
    B = hf_ref.shape[0]
    CHL = keysT_ref.shape[1]
    nlb = B // CB
    ngr = B // G
    lab = lab_ref[...]

    sumsq = jnp.int32(0)
    for c in range(NUM_CLASSES):
        n_c = jnp.sum((lab == c).astype(jnp.int32))
        sumsq = sumsq + n_c * n_c
    neg_mean = (jnp.float32(B) * jnp.float32(B) - sumsq.astype(jnp.float32)) / jnp.float32(B)
    kk = jnp.floor(jnp.float32(HARD_NEG_RATIO) * neg_mean).astype(jnp.int32)

    def lt_block(cb, g):
        return lax.dot_general(
            hf_ref[pl.ds(g * G, G), :], hmT_ref[:, pl.ds(cb * CB, CB)],
            (((1,), (0,)), ((), ())), preferred_element_type=jnp.float32) / TEMP

    def same_block(cb, g):
        return lax.dot_general(
            oh_ref[pl.ds(g * G, G), :], ohT_ref[:, pl.ds(cb * CB, CB)],
            (((1,), (0,)), ((), ())), preferred_element_type=jnp.float32)

    ones_cb = jnp.ones((CB, 1), dtype=jnp.float32)

    def p1_block(cb, _):
        def p1_group(g, carry):
            rmax, rsum = carry
            logits = lt_block(cb, g)
            gmax_r = jnp.max(logits, axis=0, keepdims=True)
            nmax = jnp.maximum(rmax, gmax_r)
            rsum = rsum * jnp.exp(rmax - nmax) + jnp.sum(
                jnp.exp(logits - nmax), axis=0, keepdims=True)

            gmax_c = jnp.max(logits, axis=1, keepdims=True)
            cold = colmax_ref[pl.ds(g * G, G), :]
            cnew = jnp.where(cb == 0, gmax_c, jnp.maximum(cold, gmax_c))
            e2 = jnp.exp(logits - cnew)
            part = lax.dot_general(e2, ones_cb, (((1,), (0,)), ((), ())),
                                   preferred_element_type=jnp.float32)
            csold = colsum_ref[pl.ds(g * G, G), :]
            colsum_ref[pl.ds(g * G, G), :] = jnp.where(
                cb == 0, part, csold * jnp.exp(cold - cnew) + part)
            colmax_ref[pl.ds(g * G, G), :] = cnew
            return nmax, rsum

        rmax0 = jnp.full((1, CB), -jnp.inf, dtype=jnp.float32)
        rsum0 = jnp.zeros((1, CB), dtype=jnp.float32)
        rmax, rsum = lax.fori_loop(0, ngr, p1_group, (rmax0, rsum0))
        rowlse_ref[:, pl.ds(cb * CB, CB)] = rmax + jnp.log(rsum)
        return 0

    lax.fori_loop(0, nlb, p1_block, 0)

    def collse_group(g, _):
        colmax_ref[pl.ds(g * G, G), :] = (
            colmax_ref[pl.ds(g * G, G), :]
            + jnp.log(colsum_ref[pl.ds(g * G, G), :]))
        return 0

    lax.fori_loop(0, ngr, collse_group, 0)

    clb = CHL // CB
    nrd = B // (8 * UNROLL)

    def p3_chunk(ch, acc):
        def write_keys(t, _):
            cb2 = t // ngr
            g = t % ngr
            logits = lt_block(ch * clb + cb2, g)
            negv = logits * (1.0 - same_block(ch * clb + cb2, g))
            bits = lax.bitcast_convert_type(negv, jnp.int32)
            m = lax.shift_right_arithmetic(bits, 31) | jnp.int32(-2147483648)
            keysT_ref[pl.ds(g * G, G), pl.ds(cb2 * CB, CB)] = (
                lax.bitcast_convert_type(bits ^ m, jnp.uint32))
            return 0

        lax.fori_loop(0, clb * ngr, write_keys, 0)

        def bis_cond(st):
            it, lo, hi, cntlo = st
            return jnp.logical_and(it < BISECT_ITERS,
                                   jnp.logical_not(jnp.all(cntlo == kk)))

        def bis_body(st):
            it, lo, hi, cntlo = st
            mid = lo + lax.shift_right_logical(hi - lo, jnp.uint32(1))

            def count_rows(r, acc8):
                base = r * 8 * UNROLL
                for u in range(UNROLL):
                    k8 = keysT_ref[pl.ds(base + u * 8, 8), :]
                    acc8 = acc8 + (k8 > mid).astype(jnp.int32)
                return acc8

            acc8 = lax.fori_loop(0, nrd, count_rows,
                                 jnp.zeros((8, CHL), dtype=jnp.int32))
            cnt = jnp.sum(acc8, axis=0, keepdims=True)
            ge = cnt >= kk
            return (it + 1, jnp.where(ge, mid, lo), jnp.where(ge, hi, mid),
                    jnp.where(ge, cnt, cntlo))

        lo0 = jnp.zeros((1, CHL), dtype=jnp.uint32)
        hi0 = jnp.full((1, CHL), jnp.uint32(0xFFFFFFFF))
        cnt0 = jnp.full((1, CHL), jnp.int32(-1))
        _, lo, _, _ = lax.while_loop(bis_cond, bis_body,
                                     (jnp.int32(0), lo0, hi0, cnt0))
        lo_ref[...] = lo

        def accum(t, acc):
            cb2 = t // ngr
            g = t % ngr
            cb = ch * clb + cb2
            same = same_block(cb, g)
            keys_g = keysT_ref[pl.ds(g * G, G), pl.ds(cb2 * CB, CB)]
            w = jnp.where(keys_g > lo_ref[:, pl.ds(cb2 * CB, CB)], 0.5, same)
            wh = lax.dot_general(w, hm_ref[pl.ds(cb * CB, CB), :],
                                 (((1,), (0,)), ((), ())),
                                 preferred_element_type=jnp.float32)
            s_wl = jnp.sum(wh * hf_ref[pl.ds(g * G, G), :])
            wcol = jnp.sum(w, axis=0, keepdims=True)
            wlane = lax.dot_general(w, ones_cb, (((1,), (0,)), ((), ())),
                                    preferred_element_type=jnp.float32)
            rlse = rowlse_ref[:, pl.ds(cb * CB, CB)]
            clse = colmax_ref[pl.ds(g * G, G), :]
            s_row = jnp.sum(wcol * rlse)
            s_col = jnp.sum(wlane * clse)
            return acc + ((2.0 / TEMP) * s_wl - s_row - s_col)

        return lax.fori_loop(0, clb * ngr, accum, acc)

    acc = lax.fori_loop(0, B // CHL, p3_chunk, jnp.float32(0.0))
    out_ref[...] = (-acc / (jnp.float32(B) * jnp.float32(B))).reshape(1, 1)


@jax.jit
def kernel(h_microbe, h_fmri, labels):
    B = h_microbe.shape[0]
    oh = (labels[:, None] == jnp.arange(NUM_CLASSES)[None, :]).astype(jnp.float32)
    lab_col = labels.reshape(1, B).astype(jnp.int32)
    out = pl.pallas_call(
        _loss_kernel,
        out_shape=jax.ShapeDtypeStruct((1, 1), jnp.float32),
        scratch_shapes=[
            pltpu.VMEM((B, min(CHUNK_L, B)), jnp.uint32),
            pltpu.VMEM((1, B), jnp.float32),
            pltpu.VMEM((B, 1), jnp.float32),
            pltpu.VMEM((B, 1), jnp.float32),
            pltpu.VMEM((1, min(CHUNK_L, B)), jnp.uint32),
        ],
    )(h_fmri, h_microbe.T, h_microbe, oh, oh.T, lab_col)
    return out[0, 0]


# speedup vs baseline: 1.0325x; 1.0325x over previous
import jax
import jax.numpy as jnp
from jax import lax
from jax.experimental import pallas as pl
from jax.experimental.pallas import tpu as pltpu

TEMP = 0.07
HARD_NEG_RATIO = 0.2
NUM_CLASSES = 10
CB = 128
G = 256
RB = 8
CHUNK_L = 1024
BISECT_ITERS = 32
UNROLL = 8


def _loss_kernel(hm_ref, hfT_ref, hf_ref, hmT_ref, oh_ref, ohT_ref, lab_ref,
                 out_ref, keysT_ref, rowlse_ref, cmaxrow_ref, csumrow_ref,
                 wlane_ref, lo_ref):
    B = hm_ref.shape[0]
    CHL = keysT_ref.shape[1]
    nrb = B // RB
    nlb = B // CB
    ngr = B // G
    lab = lab_ref[...]

    sumsq = jnp.int32(0)
    for c in range(NUM_CLASSES):
        n_c = jnp.sum((lab == c).astype(jnp.int32))
        sumsq = sumsq + n_c * n_c
    neg_mean = (jnp.float32(B) * jnp.float32(B) - sumsq.astype(jnp.float32)) / jnp.float32(B)
    kk = jnp.floor(jnp.float32(HARD_NEG_RATIO) * neg_mean).astype(jnp.int32)

    def l_row_block(rb):
        return lax.dot_general(
            hm_ref[pl.ds(rb * RB, RB), :], hfT_ref[...],
            (((1,), (0,)), ((), ())), preferred_element_type=jnp.float32) / TEMP

    def lt_block(cb, g):
        return lax.dot_general(
            hf_ref[pl.ds(g * G, G), :], hmT_ref[:, pl.ds(cb * CB, CB)],
            (((1,), (0,)), ((), ())), preferred_element_type=jnp.float32) / TEMP

    def same_block(cb, g):
        return lax.dot_general(
            oh_ref[pl.ds(g * G, G), :], ohT_ref[:, pl.ds(cb * CB, CB)],
            (((1,), (0,)), ((), ())), preferred_element_type=jnp.float32)

    def cmax_block(rb, cmax):
        return jnp.maximum(cmax, jnp.max(l_row_block(rb), axis=0, keepdims=True))

    cmax = lax.fori_loop(0, nrb, cmax_block,
                         jnp.full((1, B), -jnp.inf, dtype=jnp.float32))
    cmaxrow_ref[...] = cmax

    def csum_block(rb, csum):
        e = jnp.exp(l_row_block(rb) - cmax)
        return csum + jnp.sum(e, axis=0, keepdims=True)

    csumrow_ref[...] = lax.fori_loop(0, nrb, csum_block,
                                     jnp.zeros((1, B), dtype=jnp.float32))

    clb = CHL // CB
    nrd = B // (8 * UNROLL)
    ones_cb = jnp.ones((CB, 1), dtype=jnp.float32)

    def p3_chunk(ch, acc):
        def write_lane_block(cb2, _):
            cb = ch * clb + cb2

            def write_group(g, carry):
                rmax, rsum = carry
                logits = lt_block(cb, g)
                gmax = jnp.max(logits, axis=0, keepdims=True)
                nmax = jnp.maximum(rmax, gmax)
                rsum = rsum * jnp.exp(rmax - nmax) + jnp.sum(
                    jnp.exp(logits - nmax), axis=0, keepdims=True)
                negv = logits * (1.0 - same_block(cb, g))
                bits = lax.bitcast_convert_type(negv, jnp.int32)
                m = lax.shift_right_arithmetic(bits, 31) | jnp.int32(-2147483648)
                keysT_ref[pl.ds(g * G, G), pl.ds(cb2 * CB, CB)] = (
                    lax.bitcast_convert_type(bits ^ m, jnp.uint32))
                return nmax, rsum

            rmax0 = jnp.full((1, CB), -jnp.inf, dtype=jnp.float32)
            rsum0 = jnp.zeros((1, CB), dtype=jnp.float32)
            rmax, rsum = lax.fori_loop(0, ngr, write_group, (rmax0, rsum0))
            rowlse_ref[:, pl.ds(cb * CB, CB)] = rmax + jnp.log(rsum)
            return 0

        lax.fori_loop(0, clb, write_lane_block, 0)

        def bis_cond(st):
            it, lo, hi, cntlo = st
            return jnp.logical_and(it < BISECT_ITERS,
                                   jnp.logical_not(jnp.all(cntlo == kk)))

        def bis_body(st):
            it, lo, hi, cntlo = st
            mid = lo + lax.shift_right_logical(hi - lo, jnp.uint32(1))

            def count_rows(r, acc8):
                base = r * 8 * UNROLL
                for u in range(UNROLL):
                    k8 = keysT_ref[pl.ds(base + u * 8, 8), :]
                    acc8 = acc8 + (k8 > mid).astype(jnp.int32)
                return acc8

            acc8 = lax.fori_loop(0, nrd, count_rows,
                                 jnp.zeros((8, CHL), dtype=jnp.int32))
            cnt = jnp.sum(acc8, axis=0, keepdims=True)
            ge = cnt >= kk
            return (it + 1, jnp.where(ge, mid, lo), jnp.where(ge, hi, mid),
                    jnp.where(ge, cnt, cntlo))

        lo0 = jnp.zeros((1, CHL), dtype=jnp.uint32)
        hi0 = jnp.full((1, CHL), jnp.uint32(0xFFFFFFFF))
        cnt0 = jnp.full((1, CHL), jnp.int32(-1))
        _, lo, _, _ = lax.while_loop(bis_cond, bis_body,
                                     (jnp.int32(0), lo0, hi0, cnt0))
        lo_ref[...] = lo

        def accum(t, acc):
            cb2 = t // ngr
            g = t % ngr
            cb = ch * clb + cb2
            same = same_block(cb, g)
            keys_g = keysT_ref[pl.ds(g * G, G), pl.ds(cb2 * CB, CB)]
            w = jnp.where(keys_g > lo_ref[:, pl.ds(cb2 * CB, CB)], 0.5, same)
            wh = lax.dot_general(w, hm_ref[pl.ds(cb * CB, CB), :],
                                 (((1,), (0,)), ((), ())),
                                 preferred_element_type=jnp.float32)
            s_wl = jnp.sum(wh * hf_ref[pl.ds(g * G, G), :])
            wcol = jnp.sum(w, axis=0, keepdims=True)
            s_row = jnp.sum(wcol * rowlse_ref[:, pl.ds(cb * CB, CB)])
            wl = lax.dot_general(w, ones_cb, (((1,), (0,)), ((), ())),
                                 preferred_element_type=jnp.float32)
            old = wlane_ref[pl.ds(g * G, G), :]
            wlane_ref[pl.ds(g * G, G), :] = jnp.where(cb == 0, wl, old + wl)
            return acc + ((2.0 / TEMP) * s_wl - s_row)

        return lax.fori_loop(0, clb * ngr, accum, acc)

    acc = lax.fori_loop(0, B // CHL, p3_chunk, jnp.float32(0.0))

    clse = cmaxrow_ref[...] + jnp.log(csumrow_ref[...])
    s_col = lax.dot_general(clse, wlane_ref[...], (((1,), (0,)), ((), ())),
                            preferred_element_type=jnp.float32)
    total = acc - jnp.sum(s_col)
    out_ref[...] = (-total / (jnp.float32(B) * jnp.float32(B))).reshape(1, 1)


@jax.jit
def kernel(h_microbe, h_fmri, labels):
    B = h_microbe.shape[0]
    oh = (labels[:, None] == jnp.arange(NUM_CLASSES)[None, :]).astype(jnp.float32)
    lab_col = labels.reshape(1, B).astype(jnp.int32)
    out = pl.pallas_call(
        _loss_kernel,
        out_shape=jax.ShapeDtypeStruct((1, 1), jnp.float32),
        scratch_shapes=[
            pltpu.VMEM((B, min(CHUNK_L, B)), jnp.uint32),
            pltpu.VMEM((1, B), jnp.float32),
            pltpu.VMEM((1, B), jnp.float32),
            pltpu.VMEM((1, B), jnp.float32),
            pltpu.VMEM((B, 1), jnp.float32),
            pltpu.VMEM((1, min(CHUNK_L, B)), jnp.uint32),
        ],
    )(h_microbe, h_fmri.T, h_fmri, h_microbe.T, oh, oh.T, lab_col)
    return out[0, 0]


# speedup vs baseline: 1.1240x; 1.0886x over previous
import jax
import jax.numpy as jnp
from jax import lax
from jax.experimental import pallas as pl
from jax.experimental.pallas import tpu as pltpu

TEMP = 0.07
HARD_NEG_RATIO = 0.2
NUM_CLASSES = 10
CB = 128
G = 256
RB = 8
CHUNK_L = 1024
BISECT_ITERS = 32
UNROLL = 8


def _loss_kernel(hm_ref, hfT_ref, hf_ref, hmT_ref, oh_ref, ohT_ref, lab_ref,
                 out_ref, keysT_ref, rowlse_ref, cmaxrow_ref, csumrow_ref,
                 wlane_ref, lo_ref):
    B = hm_ref.shape[0]
    CHL = keysT_ref.shape[1]
    nrb = B // RB
    nlb = B // CB
    ngr = B // G
    lab = lab_ref[...]

    sumsq = jnp.int32(0)
    ncs = []
    for c in range(NUM_CLASSES):
        n_c = jnp.sum((lab == c).astype(jnp.int32))
        ncs.append(n_c)
        sumsq = sumsq + n_c * n_c
    neg_mean = (jnp.float32(B) * jnp.float32(B) - sumsq.astype(jnp.float32)) / jnp.float32(B)
    kk = jnp.floor(jnp.float32(HARD_NEG_RATIO) * neg_mean).astype(jnp.int32)

    def l_row_block(rb):
        return lax.dot_general(
            hm_ref[pl.ds(rb * RB, RB), :], hfT_ref[...],
            (((1,), (0,)), ((), ())), preferred_element_type=jnp.float32) / TEMP

    def lt_block(cb, g):
        return lax.dot_general(
            hf_ref[pl.ds(g * G, G), :], hmT_ref[:, pl.ds(cb * CB, CB)],
            (((1,), (0,)), ((), ())), preferred_element_type=jnp.float32) / TEMP

    def same_block(cb, g):
        return lax.dot_general(
            oh_ref[pl.ds(g * G, G), :], ohT_ref[:, pl.ds(cb * CB, CB)],
            (((1,), (0,)), ((), ())), preferred_element_type=jnp.float32)

    def cmax_block(rb, cmax):
        return jnp.maximum(cmax, jnp.max(l_row_block(rb), axis=0, keepdims=True))

    cmax = lax.fori_loop(0, nrb, cmax_block,
                         jnp.full((1, B), -jnp.inf, dtype=jnp.float32))
    cmaxrow_ref[...] = cmax

    def csum_block(rb, csum):
        e = jnp.exp(l_row_block(rb) - cmax)
        return csum + jnp.sum(e, axis=0, keepdims=True)

    csumrow_ref[...] = lax.fori_loop(0, nrb, csum_block,
                                     jnp.zeros((1, B), dtype=jnp.float32))

    clb = CHL // CB
    nrd = B // (8 * UNROLL)
    ones_cb = jnp.ones((CB, 1), dtype=jnp.float32)

    def p3_chunk(ch, acc):
        def write_lane_block(cb2, _):
            cb = ch * clb + cb2

            def write_group(g, carry):
                rmax, rsum = carry
                logits = lt_block(cb, g)
                gmax = jnp.max(logits, axis=0, keepdims=True)
                nmax = jnp.maximum(rmax, gmax)
                rsum = rsum * jnp.exp(rmax - nmax) + jnp.sum(
                    jnp.exp(logits - nmax), axis=0, keepdims=True)
                negv = logits * (1.0 - same_block(cb, g))
                bits = lax.bitcast_convert_type(negv, jnp.int32)
                m = lax.shift_right_arithmetic(bits, 31) | jnp.int32(-2147483648)
                keysT_ref[pl.ds(g * G, G), pl.ds(cb2 * CB, CB)] = (
                    lax.bitcast_convert_type(bits ^ m, jnp.uint32))
                return nmax, rsum

            rmax0 = jnp.full((1, CB), -jnp.inf, dtype=jnp.float32)
            rsum0 = jnp.zeros((1, CB), dtype=jnp.float32)
            rmax, rsum = lax.fori_loop(0, ngr, write_group, (rmax0, rsum0))
            rowlse_ref[:, pl.ds(cb * CB, CB)] = rmax + jnp.log(rsum)
            return 0

        lax.fori_loop(0, clb, write_lane_block, 0)

        def bis_cond(st):
            it, lo, hi, cntlo = st
            return jnp.logical_and(it < BISECT_ITERS,
                                   jnp.logical_not(jnp.all(cntlo == kk)))

        def bis_body(st):
            it, lo, hi, cntlo = st
            mid = lo + lax.shift_right_logical(hi - lo, jnp.uint32(1))

            def count_rows(r, acc8):
                base = r * 8 * UNROLL
                for u in range(UNROLL):
                    k8 = keysT_ref[pl.ds(base + u * 8, 8), :]
                    acc8 = jnp.where(k8 > mid, acc8 + 1, acc8)
                return acc8

            acc8 = lax.fori_loop(0, nrd, count_rows,
                                 jnp.zeros((8, CHL), dtype=jnp.int32))
            cnt = jnp.sum(acc8, axis=0, keepdims=True)
            ge = cnt >= kk
            return (it + 1, jnp.where(ge, mid, lo), jnp.where(ge, hi, mid),
                    jnp.where(ge, cnt, cntlo))

        lo0 = jnp.zeros((1, CHL), dtype=jnp.uint32)
        hi0 = jnp.full((1, CHL), jnp.uint32(0xFFFFFFFF))
        cnt0 = jnp.full((1, CHL), jnp.int32(-1))
        _, lo, _, cntlo = lax.while_loop(bis_cond, bis_body,
                                         (jnp.int32(0), lo0, hi0, cnt0))
        lo_ref[...] = lo

        s_row_m = 0.5 * jnp.sum(cntlo.astype(jnp.float32)
                                * rowlse_ref[:, pl.ds(ch * CHL, CHL)])

        def accum(t, acc):
            cb2 = t // ngr
            g = t % ngr
            cb = ch * clb + cb2
            keys_g = keysT_ref[pl.ds(g * G, G), pl.ds(cb2 * CB, CB)]
            w = jnp.where(keys_g > lo_ref[:, pl.ds(cb2 * CB, CB)], 0.5, 0.0)
            wh = lax.dot_general(w, hm_ref[pl.ds(cb * CB, CB), :],
                                 (((1,), (0,)), ((), ())),
                                 preferred_element_type=jnp.float32)
            s_wl = jnp.sum(wh * hf_ref[pl.ds(g * G, G), :])
            wl = lax.dot_general(w, ones_cb, (((1,), (0,)), ((), ())),
                                 preferred_element_type=jnp.float32)
            old = wlane_ref[pl.ds(g * G, G), :]
            wlane_ref[pl.ds(g * G, G), :] = jnp.where(cb == 0, wl, old + wl)
            return acc + (2.0 / TEMP) * s_wl

        acc = lax.fori_loop(0, clb * ngr, accum, acc) - s_row_m

        has_ms = jnp.any(lo < jnp.uint32(0x80000000))

        def ms_correct(t, acc):
            cb2 = t // ngr
            g = t % ngr
            cb = ch * clb + cb2
            same = same_block(cb, g)
            keys_g = keysT_ref[pl.ds(g * G, G), pl.ds(cb2 * CB, CB)]
            marked = keys_g > lo_ref[:, pl.ds(cb2 * CB, CB)]
            ms = jnp.where(marked, same, 0.0)
            mh = lax.dot_general(ms, hm_ref[pl.ds(cb * CB, CB), :],
                                 (((1,), (0,)), ((), ())),
                                 preferred_element_type=jnp.float32)
            s_ml = jnp.sum(mh * hf_ref[pl.ds(g * G, G), :])
            mcol = jnp.sum(ms, axis=0, keepdims=True)
            s_mrow = jnp.sum(mcol * rowlse_ref[:, pl.ds(cb * CB, CB)])
            ml = lax.dot_general(ms, ones_cb, (((1,), (0,)), ((), ())),
                                 preferred_element_type=jnp.float32)
            old = wlane_ref[pl.ds(g * G, G), :]
            wlane_ref[pl.ds(g * G, G), :] = old - ml
            return acc - (2.0 / TEMP) * s_ml + s_mrow

        return lax.cond(has_ms,
                        lambda a: lax.fori_loop(0, clb * ngr, ms_correct, a),
                        lambda a: a, acc)

    acc = lax.fori_loop(0, B // CHL, p3_chunk, jnp.float32(0.0))

    clse = cmaxrow_ref[...] + jnp.log(csumrow_ref[...])
    s_col = lax.dot_general(clse, wlane_ref[...], (((1,), (0,)), ((), ())),
                            preferred_element_type=jnp.float32)

    ohm = lax.dot_general(ohT_ref[...], hm_ref[...], (((1,), (0,)), ((), ())),
                          preferred_element_type=jnp.float32)
    ohf = lax.dot_general(ohT_ref[...], hf_ref[...], (((1,), (0,)), ((), ())),
                          preferred_element_type=jnp.float32)
    s_same_l = jnp.sum(ohm * ohf) / TEMP
    nvec = jnp.zeros((1, B), dtype=jnp.float32)
    for c in range(NUM_CLASSES):
        nvec = jnp.where(lab == c, ncs[c].astype(jnp.float32), nvec)
    s_same_row = jnp.sum(nvec * rowlse_ref[...])
    s_same_col = jnp.sum(nvec * clse)

    total = acc + 2.0 * s_same_l - s_same_row - s_same_col - jnp.sum(s_col)
    out_ref[...] = (-total / (jnp.float32(B) * jnp.float32(B))).reshape(1, 1)


@jax.jit
def kernel(h_microbe, h_fmri, labels):
    B = h_microbe.shape[0]
    oh = (labels[:, None] == jnp.arange(NUM_CLASSES)[None, :]).astype(jnp.float32)
    lab_col = labels.reshape(1, B).astype(jnp.int32)
    out = pl.pallas_call(
        _loss_kernel,
        out_shape=jax.ShapeDtypeStruct((1, 1), jnp.float32),
        scratch_shapes=[
            pltpu.VMEM((B, min(CHUNK_L, B)), jnp.uint32),
            pltpu.VMEM((1, B), jnp.float32),
            pltpu.VMEM((1, B), jnp.float32),
            pltpu.VMEM((1, B), jnp.float32),
            pltpu.VMEM((B, 1), jnp.float32),
            pltpu.VMEM((1, min(CHUNK_L, B)), jnp.uint32),
        ],
    )(h_microbe, h_fmri.T, h_fmri, h_microbe.T, oh, oh.T, lab_col)
    return out[0, 0]


# speedup vs baseline: 1.1982x; 1.0659x over previous
import jax
import jax.numpy as jnp
from jax import lax
from jax.experimental import pallas as pl
from jax.experimental.pallas import tpu as pltpu

TEMP = 0.07
HARD_NEG_RATIO = 0.2
NUM_CLASSES = 10
CB = 128
G = 256
RB = 8
CHUNK_L = 1024
BISECT_ITERS = 32
UNROLL = 64
U16 = 32


def _loss_kernel(hm_ref, hfT_ref, hf_ref, hmT_ref, oh_ref, ohT_ref, lab_ref,
                 out_ref, keysT_ref, keys16_ref, rowlse_ref, cmaxrow_ref,
                 csumrow_ref, wlane_ref, lo_ref):
    B = hm_ref.shape[0]
    CHL = keysT_ref.shape[1]
    nrb = B // RB
    nlb = B // CB
    ngr = B // G
    lab = lab_ref[...]

    sumsq = jnp.int32(0)
    ncs = []
    for c in range(NUM_CLASSES):
        n_c = jnp.sum((lab == c).astype(jnp.int32))
        ncs.append(n_c)
        sumsq = sumsq + n_c * n_c
    neg_mean = (jnp.float32(B) * jnp.float32(B) - sumsq.astype(jnp.float32)) / jnp.float32(B)
    kk = jnp.floor(jnp.float32(HARD_NEG_RATIO) * neg_mean).astype(jnp.int32)

    def l_row_block(rb):
        return lax.dot_general(
            hm_ref[pl.ds(rb * RB, RB), :], hfT_ref[...],
            (((1,), (0,)), ((), ())), preferred_element_type=jnp.float32) / TEMP

    def lt_block(cb, g):
        return lax.dot_general(
            hf_ref[pl.ds(g * G, G), :], hmT_ref[:, pl.ds(cb * CB, CB)],
            (((1,), (0,)), ((), ())), preferred_element_type=jnp.float32) / TEMP

    def same_block(cb, g):
        return lax.dot_general(
            oh_ref[pl.ds(g * G, G), :], ohT_ref[:, pl.ds(cb * CB, CB)],
            (((1,), (0,)), ((), ())), preferred_element_type=jnp.float32)

    def cmax_block(rb, cmax):
        return jnp.maximum(cmax, jnp.max(l_row_block(rb), axis=0, keepdims=True))

    cmax = lax.fori_loop(0, nrb, cmax_block,
                         jnp.full((1, B), -jnp.inf, dtype=jnp.float32))
    cmaxrow_ref[...] = cmax

    def csum_block(rb, csum):
        e = jnp.exp(l_row_block(rb) - cmax)
        return csum + jnp.sum(e, axis=0, keepdims=True)

    csumrow_ref[...] = lax.fori_loop(0, nrb, csum_block,
                                     jnp.zeros((1, B), dtype=jnp.float32))

    clb = CHL // CB
    nrd = B // (8 * UNROLL)
    nrd16 = B // (16 * U16)
    ones_cb = jnp.ones((CB, 1), dtype=jnp.float32)

    def p3_chunk(ch, acc):
        def write_lane_block(cb2, _):
            cb = ch * clb + cb2

            def write_group(g, carry):
                rmax, rsum = carry
                logits = lt_block(cb, g)
                gmax = jnp.max(logits, axis=0, keepdims=True)
                nmax = jnp.maximum(rmax, gmax)
                rsum = rsum * jnp.exp(rmax - nmax) + jnp.sum(
                    jnp.exp(logits - nmax), axis=0, keepdims=True)
                negv = logits * (1.0 - same_block(cb, g))
                bits = lax.bitcast_convert_type(negv, jnp.int32)
                m = lax.shift_right_arithmetic(bits, 31) | jnp.int32(-2147483648)
                ukey = lax.bitcast_convert_type(bits ^ m, jnp.uint32)
                keysT_ref[pl.ds(g * G, G), pl.ds(cb2 * CB, CB)] = ukey
                h16 = lax.shift_right_logical(ukey, jnp.uint32(16))
                keys16_ref[pl.ds(g * G, G), pl.ds(cb2 * CB, CB)] = (
                    (h16 ^ jnp.uint32(0x8000)).astype(jnp.int32)
                    .astype(jnp.int16))
                return nmax, rsum

            rmax0 = jnp.full((1, CB), -jnp.inf, dtype=jnp.float32)
            rsum0 = jnp.zeros((1, CB), dtype=jnp.float32)
            rmax, rsum = lax.fori_loop(0, ngr, write_group, (rmax0, rsum0))
            rowlse_ref[:, pl.ds(cb * CB, CB)] = rmax + jnp.log(rsum)
            return 0

        lax.fori_loop(0, clb, write_lane_block, 0)

        def bis_cond(st):
            it, lo, hi, cntlo = st
            return jnp.logical_and(it < BISECT_ITERS,
                                   jnp.logical_not(jnp.all(cntlo == kk)))

        def bis_body(st):
            it, lo, hi, cntlo = st
            mid = lo + lax.shift_right_logical(hi - lo, jnp.uint32(1))

            def count_rows(r, acc8):
                base = r * 8 * UNROLL
                for u in range(UNROLL):
                    k8 = keysT_ref[pl.ds(base + u * 8, 8), :]
                    acc8 = jnp.where(k8 > mid, acc8 + 1, acc8)
                return acc8

            acc8 = lax.fori_loop(0, nrd, count_rows,
                                 jnp.zeros((8, CHL), dtype=jnp.int32))
            cnt = jnp.sum(acc8, axis=0, keepdims=True)
            ge = cnt >= kk
            return (it + 1, jnp.where(ge, mid, lo), jnp.where(ge, hi, mid),
                    jnp.where(ge, cnt, cntlo))

        def bis16_cond(st):
            it, lo, hi, cntlo = st
            return jnp.logical_and(it < 16,
                                   jnp.logical_not(jnp.all(cntlo == kk)))

        def bis16_body(st):
            it, lo, hi, cntlo = st
            mid = lo + lax.shift_right_logical(hi - lo, jnp.uint32(1))
            mid16 = (mid ^ jnp.uint32(0x8000)).astype(jnp.int32).astype(jnp.int16)

            def count16(r, acc16):
                base = r * 16 * U16
                for u in range(U16):
                    k16 = keys16_ref[pl.ds(base + u * 16, 16), :]
                    acc16 = jnp.where(k16 > mid16, acc16 + 1, acc16)
                return acc16

            acc16 = lax.fori_loop(0, nrd16, count16,
                                  jnp.zeros((16, CHL), dtype=jnp.int16))
            cnt = jnp.sum(acc16.astype(jnp.int32), axis=0, keepdims=True)
            ge = cnt >= kk
            return (it + 1, jnp.where(ge, mid, lo), jnp.where(ge, hi, mid),
                    jnp.where(ge, cnt, cntlo))

        lo16_0 = jnp.zeros((1, CHL), dtype=jnp.uint32)
        hi16_0 = jnp.full((1, CHL), jnp.uint32(0xFFFF))
        cnt16_0 = jnp.full((1, CHL), jnp.int32(-1))
        _, lo16, hi16, cnt16 = lax.while_loop(
            bis16_cond, bis16_body, (jnp.int32(0), lo16_0, hi16_0, cnt16_0))

        lo0 = (lo16 << jnp.uint32(16)) | jnp.uint32(0xFFFF)
        hi0 = (hi16 << jnp.uint32(16)) | jnp.uint32(0xFFFF)
        _, lo, _, cntlo = lax.while_loop(bis_cond, bis_body,
                                         (jnp.int32(0), lo0, hi0, cnt16))
        lo_ref[...] = lo

        s_row_m = 0.5 * jnp.sum(cntlo.astype(jnp.float32)
                                * rowlse_ref[:, pl.ds(ch * CHL, CHL)])

        def accum(t, acc):
            cb2 = t // ngr
            g = t % ngr
            cb = ch * clb + cb2
            keys_g = keysT_ref[pl.ds(g * G, G), pl.ds(cb2 * CB, CB)]
            w = jnp.where(keys_g > lo_ref[:, pl.ds(cb2 * CB, CB)], 0.5, 0.0)
            wh = lax.dot_general(w, hm_ref[pl.ds(cb * CB, CB), :],
                                 (((1,), (0,)), ((), ())),
                                 preferred_element_type=jnp.float32)
            s_wl = jnp.sum(wh * hf_ref[pl.ds(g * G, G), :])
            wl = lax.dot_general(w, ones_cb, (((1,), (0,)), ((), ())),
                                 preferred_element_type=jnp.float32)
            old = wlane_ref[pl.ds(g * G, G), :]
            wlane_ref[pl.ds(g * G, G), :] = jnp.where(cb == 0, wl, old + wl)
            return acc + (2.0 / TEMP) * s_wl

        acc = lax.fori_loop(0, clb * ngr, accum, acc) - s_row_m

        has_ms = jnp.any(lo < jnp.uint32(0x80000000))

        def ms_correct(t, acc):
            cb2 = t // ngr
            g = t % ngr
            cb = ch * clb + cb2
            same = same_block(cb, g)
            keys_g = keysT_ref[pl.ds(g * G, G), pl.ds(cb2 * CB, CB)]
            marked = keys_g > lo_ref[:, pl.ds(cb2 * CB, CB)]
            ms = jnp.where(marked, same, 0.0)
            mh = lax.dot_general(ms, hm_ref[pl.ds(cb * CB, CB), :],
                                 (((1,), (0,)), ((), ())),
                                 preferred_element_type=jnp.float32)
            s_ml = jnp.sum(mh * hf_ref[pl.ds(g * G, G), :])
            mcol = jnp.sum(ms, axis=0, keepdims=True)
            s_mrow = jnp.sum(mcol * rowlse_ref[:, pl.ds(cb * CB, CB)])
            ml = lax.dot_general(ms, ones_cb, (((1,), (0,)), ((), ())),
                                 preferred_element_type=jnp.float32)
            old = wlane_ref[pl.ds(g * G, G), :]
            wlane_ref[pl.ds(g * G, G), :] = old - ml
            return acc - (2.0 / TEMP) * s_ml + s_mrow

        return lax.cond(has_ms,
                        lambda a: lax.fori_loop(0, clb * ngr, ms_correct, a),
                        lambda a: a, acc)

    acc = lax.fori_loop(0, B // CHL, p3_chunk, jnp.float32(0.0))

    clse = cmaxrow_ref[...] + jnp.log(csumrow_ref[...])
    s_col = lax.dot_general(clse, wlane_ref[...], (((1,), (0,)), ((), ())),
                            preferred_element_type=jnp.float32)

    ohm = lax.dot_general(ohT_ref[...], hm_ref[...], (((1,), (0,)), ((), ())),
                          preferred_element_type=jnp.float32)
    ohf = lax.dot_general(ohT_ref[...], hf_ref[...], (((1,), (0,)), ((), ())),
                          preferred_element_type=jnp.float32)
    s_same_l = jnp.sum(ohm * ohf) / TEMP
    nvec = jnp.zeros((1, B), dtype=jnp.float32)
    for c in range(NUM_CLASSES):
        nvec = jnp.where(lab == c, ncs[c].astype(jnp.float32), nvec)
    s_same_row = jnp.sum(nvec * rowlse_ref[...])
    s_same_col = jnp.sum(nvec * clse)

    total = acc + 2.0 * s_same_l - s_same_row - s_same_col - jnp.sum(s_col)
    out_ref[...] = (-total / (jnp.float32(B) * jnp.float32(B))).reshape(1, 1)


@jax.jit
def kernel(h_microbe, h_fmri, labels):
    B = h_microbe.shape[0]
    oh = (labels[:, None] == jnp.arange(NUM_CLASSES)[None, :]).astype(jnp.float32)
    lab_col = labels.reshape(1, B).astype(jnp.int32)
    out = pl.pallas_call(
        _loss_kernel,
        out_shape=jax.ShapeDtypeStruct((1, 1), jnp.float32),
        scratch_shapes=[
            pltpu.VMEM((B, min(CHUNK_L, B)), jnp.uint32),
            pltpu.VMEM((B, min(CHUNK_L, B)), jnp.int16),
            pltpu.VMEM((1, B), jnp.float32),
            pltpu.VMEM((1, B), jnp.float32),
            pltpu.VMEM((1, B), jnp.float32),
            pltpu.VMEM((B, 1), jnp.float32),
            pltpu.VMEM((1, min(CHUNK_L, B)), jnp.uint32),
        ],
    )(h_microbe, h_fmri.T, h_fmri, h_microbe.T, oh, oh.T, lab_col)
    return out[0, 0]


# speedup vs baseline: 1.1993x; 1.0009x over previous
import jax
import jax.numpy as jnp
from jax import lax
from jax.experimental import pallas as pl
from jax.experimental.pallas import tpu as pltpu

TEMP = 0.07
HARD_NEG_RATIO = 0.2
NUM_CLASSES = 10
CB = 128
G = 256
RB = 8
CHUNK_L = 1024
BISECT_ITERS = 32
UNROLL = 64
U16 = 64


def _loss_kernel(hm_ref, hfT_ref, hf_ref, hmT_ref, oh_ref, ohT_ref, lab_ref,
                 out_ref, keysT_ref, keys16_ref, rowlse_ref, cmaxrow_ref,
                 csumrow_ref, wlane_ref, lo_ref):
    B = hm_ref.shape[0]
    CHL = keysT_ref.shape[1]
    nrb = B // RB
    nlb = B // CB
    ngr = B // G
    lab = lab_ref[...]

    sumsq = jnp.int32(0)
    ncs = []
    for c in range(NUM_CLASSES):
        n_c = jnp.sum((lab == c).astype(jnp.int32))
        ncs.append(n_c)
        sumsq = sumsq + n_c * n_c
    neg_mean = (jnp.float32(B) * jnp.float32(B) - sumsq.astype(jnp.float32)) / jnp.float32(B)
    kk = jnp.floor(jnp.float32(HARD_NEG_RATIO) * neg_mean).astype(jnp.int32)

    def l_row_block(rb):
        return lax.dot_general(
            hm_ref[pl.ds(rb * RB, RB), :], hfT_ref[...],
            (((1,), (0,)), ((), ())), preferred_element_type=jnp.float32) / TEMP

    def lt_block(cb, g):
        return lax.dot_general(
            hf_ref[pl.ds(g * G, G), :], hmT_ref[:, pl.ds(cb * CB, CB)],
            (((1,), (0,)), ((), ())), preferred_element_type=jnp.float32) / TEMP

    def same_block(cb, g):
        return lax.dot_general(
            oh_ref[pl.ds(g * G, G), :], ohT_ref[:, pl.ds(cb * CB, CB)],
            (((1,), (0,)), ((), ())), preferred_element_type=jnp.float32)

    def cmax_block(rb, cmax):
        return jnp.maximum(cmax, jnp.max(l_row_block(rb), axis=0, keepdims=True))

    cmax = lax.fori_loop(0, nrb, cmax_block,
                         jnp.full((1, B), -jnp.inf, dtype=jnp.float32))
    cmaxrow_ref[...] = cmax

    def csum_block(rb, csum):
        e = jnp.exp(l_row_block(rb) - cmax)
        return csum + jnp.sum(e, axis=0, keepdims=True)

    csumrow_ref[...] = lax.fori_loop(0, nrb, csum_block,
                                     jnp.zeros((1, B), dtype=jnp.float32))

    clb = CHL // CB
    nrd = B // (8 * UNROLL)
    nrd16 = B // (16 * U16)
    ones_cb = jnp.ones((CB, 1), dtype=jnp.float32)

    def p3_chunk(ch, acc):
        def write_lane_block(cb2, _):
            cb = ch * clb + cb2

            def write_group(g, carry):
                rmax, rsum = carry
                logits = lt_block(cb, g)
                gmax = jnp.max(logits, axis=0, keepdims=True)
                nmax = jnp.maximum(rmax, gmax)
                rsum = rsum * jnp.exp(rmax - nmax) + jnp.sum(
                    jnp.exp(logits - nmax), axis=0, keepdims=True)
                negv = logits * (1.0 - same_block(cb, g))
                bits = lax.bitcast_convert_type(negv, jnp.int32)
                m = lax.shift_right_arithmetic(bits, 31) | jnp.int32(-2147483648)
                ukey = lax.bitcast_convert_type(bits ^ m, jnp.uint32)
                keysT_ref[pl.ds(g * G, G), pl.ds(cb2 * CB, CB)] = ukey
                h16 = lax.shift_right_logical(ukey, jnp.uint32(16))
                keys16_ref[pl.ds(g * G, G), pl.ds(cb2 * CB, CB)] = (
                    (h16 ^ jnp.uint32(0x8000)).astype(jnp.int32)
                    .astype(jnp.int16))
                return nmax, rsum

            rmax0 = jnp.full((1, CB), -jnp.inf, dtype=jnp.float32)
            rsum0 = jnp.zeros((1, CB), dtype=jnp.float32)
            rmax, rsum = lax.fori_loop(0, ngr, write_group, (rmax0, rsum0))
            rowlse_ref[:, pl.ds(cb * CB, CB)] = rmax + jnp.log(rsum)
            return 0

        lax.fori_loop(0, clb, write_lane_block, 0)

        def bis_cond(st):
            it, lo, hi, cntlo = st
            return jnp.logical_and(it < BISECT_ITERS,
                                   jnp.logical_not(jnp.all(cntlo == kk)))

        def bis_body(st):
            it, lo, hi, cntlo = st
            mid = lo + lax.shift_right_logical(hi - lo, jnp.uint32(1))

            def count_rows(r, acc8):
                base = r * 8 * UNROLL
                for u in range(UNROLL):
                    k8 = keysT_ref[pl.ds(base + u * 8, 8), :]
                    acc8 = jnp.where(k8 > mid, acc8 + 1, acc8)
                return acc8

            acc8 = lax.fori_loop(0, nrd, count_rows,
                                 jnp.zeros((8, CHL), dtype=jnp.int32))
            cnt = jnp.sum(acc8, axis=0, keepdims=True)
            ge = cnt >= kk
            return (it + 1, jnp.where(ge, mid, lo), jnp.where(ge, hi, mid),
                    jnp.where(ge, cnt, cntlo))

        def bis16_cond(st):
            it, lo, hi, cntlo = st
            return jnp.logical_and(it < 16,
                                   jnp.logical_not(jnp.all(cntlo == kk)))

        def bis16_body(st):
            it, lo, hi, cntlo = st
            mid = lo + lax.shift_right_logical(hi - lo, jnp.uint32(1))
            mid16 = (mid ^ jnp.uint32(0x8000)).astype(jnp.int32).astype(jnp.int16)

            def count16(r, acc16):
                base = r * 16 * U16
                for u in range(U16):
                    k16 = keys16_ref[pl.ds(base + u * 16, 16), :]
                    acc16 = jnp.where(k16 > mid16, acc16 + 1, acc16)
                return acc16

            acc16 = lax.fori_loop(0, nrd16, count16,
                                  jnp.zeros((16, CHL), dtype=jnp.int16))
            cnt = jnp.sum(acc16.astype(jnp.int32), axis=0, keepdims=True)
            ge = cnt >= kk
            return (it + 1, jnp.where(ge, mid, lo), jnp.where(ge, hi, mid),
                    jnp.where(ge, cnt, cntlo))

        lo16_0 = jnp.zeros((1, CHL), dtype=jnp.uint32)
        hi16_0 = jnp.full((1, CHL), jnp.uint32(0xFFFF))
        cnt16_0 = jnp.full((1, CHL), jnp.int32(-1))
        _, lo16, hi16, cnt16 = lax.while_loop(
            bis16_cond, bis16_body, (jnp.int32(0), lo16_0, hi16_0, cnt16_0))

        lo0 = (lo16 << jnp.uint32(16)) | jnp.uint32(0xFFFF)
        hi0 = (hi16 << jnp.uint32(16)) | jnp.uint32(0xFFFF)
        _, lo, _, cntlo = lax.while_loop(bis_cond, bis_body,
                                         (jnp.int32(0), lo0, hi0, cnt16))
        lo_ref[...] = lo

        s_row_m = 0.5 * jnp.sum(cntlo.astype(jnp.float32)
                                * rowlse_ref[:, pl.ds(ch * CHL, CHL)])

        def accum(t, acc):
            cb2 = t // ngr
            g = t % ngr
            cb = ch * clb + cb2
            keys_g = keysT_ref[pl.ds(g * G, G), pl.ds(cb2 * CB, CB)]
            w = jnp.where(keys_g > lo_ref[:, pl.ds(cb2 * CB, CB)], 0.5, 0.0)
            wh = lax.dot_general(w, hm_ref[pl.ds(cb * CB, CB), :],
                                 (((1,), (0,)), ((), ())),
                                 preferred_element_type=jnp.float32)
            s_wl = jnp.sum(wh * hf_ref[pl.ds(g * G, G), :])
            wl = lax.dot_general(w, ones_cb, (((1,), (0,)), ((), ())),
                                 preferred_element_type=jnp.float32)
            old = wlane_ref[pl.ds(g * G, G), :]
            wlane_ref[pl.ds(g * G, G), :] = jnp.where(cb == 0, wl, old + wl)
            return acc + (2.0 / TEMP) * s_wl

        acc = lax.fori_loop(0, clb * ngr, accum, acc) - s_row_m

        has_ms = jnp.any(lo < jnp.uint32(0x80000000))

        def ms_correct(t, acc):
            cb2 = t // ngr
            g = t % ngr
            cb = ch * clb + cb2
            same = same_block(cb, g)
            keys_g = keysT_ref[pl.ds(g * G, G), pl.ds(cb2 * CB, CB)]
            marked = keys_g > lo_ref[:, pl.ds(cb2 * CB, CB)]
            ms = jnp.where(marked, same, 0.0)
            mh = lax.dot_general(ms, hm_ref[pl.ds(cb * CB, CB), :],
                                 (((1,), (0,)), ((), ())),
                                 preferred_element_type=jnp.float32)
            s_ml = jnp.sum(mh * hf_ref[pl.ds(g * G, G), :])
            mcol = jnp.sum(ms, axis=0, keepdims=True)
            s_mrow = jnp.sum(mcol * rowlse_ref[:, pl.ds(cb * CB, CB)])
            ml = lax.dot_general(ms, ones_cb, (((1,), (0,)), ((), ())),
                                 preferred_element_type=jnp.float32)
            old = wlane_ref[pl.ds(g * G, G), :]
            wlane_ref[pl.ds(g * G, G), :] = old - ml
            return acc - (2.0 / TEMP) * s_ml + s_mrow

        return lax.cond(has_ms,
                        lambda a: lax.fori_loop(0, clb * ngr, ms_correct, a),
                        lambda a: a, acc)

    acc = lax.fori_loop(0, B // CHL, p3_chunk, jnp.float32(0.0))

    clse = cmaxrow_ref[...] + jnp.log(csumrow_ref[...])
    s_col = lax.dot_general(clse, wlane_ref[...], (((1,), (0,)), ((), ())),
                            preferred_element_type=jnp.float32)

    ohm = lax.dot_general(ohT_ref[...], hm_ref[...], (((1,), (0,)), ((), ())),
                          preferred_element_type=jnp.float32)
    ohf = lax.dot_general(ohT_ref[...], hf_ref[...], (((1,), (0,)), ((), ())),
                          preferred_element_type=jnp.float32)
    s_same_l = jnp.sum(ohm * ohf) / TEMP
    nvec = jnp.zeros((1, B), dtype=jnp.float32)
    for c in range(NUM_CLASSES):
        nvec = jnp.where(lab == c, ncs[c].astype(jnp.float32), nvec)
    s_same_row = jnp.sum(nvec * rowlse_ref[...])
    s_same_col = jnp.sum(nvec * clse)

    total = acc + 2.0 * s_same_l - s_same_row - s_same_col - jnp.sum(s_col)
    out_ref[...] = (-total / (jnp.float32(B) * jnp.float32(B))).reshape(1, 1)


@jax.jit
def kernel(h_microbe, h_fmri, labels):
    B = h_microbe.shape[0]
    oh = (labels[:, None] == jnp.arange(NUM_CLASSES)[None, :]).astype(jnp.float32)
    lab_col = labels.reshape(1, B).astype(jnp.int32)
    out = pl.pallas_call(
        _loss_kernel,
        out_shape=jax.ShapeDtypeStruct((1, 1), jnp.float32),
        scratch_shapes=[
            pltpu.VMEM((B, min(CHUNK_L, B)), jnp.uint32),
            pltpu.VMEM((B, min(CHUNK_L, B)), jnp.int16),
            pltpu.VMEM((1, B), jnp.float32),
            pltpu.VMEM((1, B), jnp.float32),
            pltpu.VMEM((1, B), jnp.float32),
            pltpu.VMEM((B, 1), jnp.float32),
            pltpu.VMEM((1, min(CHUNK_L, B)), jnp.uint32),
        ],
    )(h_microbe, h_fmri.T, h_fmri, h_microbe.T, oh, oh.T, lab_col)
    return out[0, 0]
